# baseline (speedup 1.0000x reference)
import functools

import jax
import jax.numpy as jnp
from jax.experimental import pallas as pl
from jax.experimental.pallas import tpu as pltpu


def _fused_kernel(x_ref, w1t_ref, b1_ref, w2t_ref, b2_ref,
                  probs_ref, aux_ref, disp_ref, comb_ref,
                  oh0_s, oh1_s, ct1_s, ct2_s, av_s, ps_s,
                  z_s, hd_s, hc_s, tail_sem, head_sem,
                  *, nr, t1, tw, cap):
    i = pl.program_id(0)
    L = 128
    nw = pl.num_programs(0) - nr

    @pl.when(i == 0)
    def _zinit():
        z_s[...] = jnp.zeros_like(z_s)

    @pl.when(i < nr)
    def _route():
        x = x_ref[...]
        h = jnp.dot(x, w1t_ref[...], preferred_element_type=jnp.float32)
        h = jnp.maximum(h + b1_ref[...], 0.0)
        logits = jnp.dot(h, w2t_ref[...], preferred_element_type=jnp.float32)
        logits = logits + b2_ref[...]

        m = jnp.max(logits, axis=-1, keepdims=True)
        ex = jnp.exp(logits - m)
        probs = ex / jnp.sum(ex, axis=-1, keepdims=True)
        probs_ref[...] = probs

        S, E = probs.shape
        iota = jax.lax.broadcasted_iota(jnp.int32, (S, E), 1)
        e0 = jnp.argmax(probs, axis=-1)
        oh0 = iota == e0[:, None]
        p0 = jnp.max(probs, axis=-1, keepdims=True)
        masked = jnp.where(oh0, -1.0, probs)
        e1 = jnp.argmax(masked, axis=-1)
        oh1 = iota == e1[:, None]
        p1 = jnp.max(masked, axis=-1, keepdims=True)
        tot = p0 + p1

        oh0f = oh0.astype(jnp.float32)
        oh1f = oh1.astype(jnp.float32)
        rows = pl.ds(i * t1, t1)
        oh0_s[rows, :] = oh0f
        oh1_s[rows, :] = oh1f
        ct1_s[rows, :] = oh0f * (p0 / tot)
        ct2_s[rows, :] = oh1f * (p1 / tot)

        blk_a = jnp.max(oh0f, axis=0, keepdims=True)
        blk_s = jnp.sum(probs, axis=0, keepdims=True)

        @pl.when(i == 0)
        def _init():
            av_s[...] = blk_a
            ps_s[...] = blk_s

        @pl.when(i > 0)
        def _acc():
            av_s[...] = jnp.maximum(av_s[...], blk_a)
            ps_s[...] = ps_s[...] + blk_s

        @pl.when(i == nr - 1)
        def _fin():
            mean_probs = ps_s[...] / (t1 * nr)
            aux_ref[...] = jnp.sum(
                mean_probs * jnp.log(mean_probs * E + 1e-9),
                axis=-1, keepdims=True)

    @pl.when(jnp.logical_and(i >= 1, i <= nr))
    def _tail_start():
        w0 = (i - 1) * 2
        for j in range(2):
            w = w0 + j
            rows = pl.ds(w * tw, tw)
            pltpu.make_async_copy(
                z_s, disp_ref.at[rows, :, :], tail_sem.at[w]).start()
            pltpu.make_async_copy(
                z_s, comb_ref.at[rows, :, :], tail_sem.at[w]).start()

    @pl.when(i >= nr)
    def _write():
        w = i - nr
        trows = pl.ds(w * tw, tw)
        pltpu.make_async_copy(
            z_s, disp_ref.at[trows, :, :], tail_sem.at[w]).wait()
        pltpu.make_async_copy(
            z_s, comb_ref.at[trows, :, :], tail_sem.at[w]).wait()
        p = jax.lax.rem(w, 2)
        rows = pl.ds(w * tw, tw)
        a = av_s[...]
        oh0 = oh0_s[rows, :]
        oh1 = oh1_s[rows, :]
        ct1 = ct1_s[rows, :]
        ct2 = ct2_s[rows, :]
        sec1 = oh1 * a
        d0 = (oh0 + (oh1 - sec1))[:, :, None]
        d1 = sec1[:, :, None]
        c1b = ct2 * a
        c0 = (ct1 + (ct2 - c1b))[:, :, None]
        c1 = c1b[:, :, None]
        ci = jax.lax.broadcasted_iota(jnp.int32, (tw, 8, L), 2)
        is0 = ci == 0
        is1 = ci == 1
        hd_s[p] = jnp.where(is0, d0, jnp.where(is1, d1, 0.0))
        hc_s[p] = jnp.where(is0, c0, jnp.where(is1, c1, 0.0))
        lanes = pl.ds(0, L)
        pltpu.make_async_copy(
            hd_s.at[p], disp_ref.at[rows, :, lanes], head_sem).start()
        pltpu.make_async_copy(
            hc_s.at[p], comb_ref.at[rows, :, lanes], head_sem).start()

        @pl.when(w > 0)
        def _head_wait_prev():
            q = jax.lax.rem(w - 1, 2)
            prows = pl.ds((w - 1) * tw, tw)
            pltpu.make_async_copy(
                hd_s.at[q], disp_ref.at[prows, :, lanes], head_sem).wait()
            pltpu.make_async_copy(
                hc_s.at[q], comb_ref.at[prows, :, lanes], head_sem).wait()

        @pl.when(w == nw - 1)
        def _head_wait_last():
            pltpu.make_async_copy(
                hd_s.at[p], disp_ref.at[rows, :, lanes], head_sem).wait()
            pltpu.make_async_copy(
                hc_s.at[p], comb_ref.at[rows, :, lanes], head_sem).wait()


def kernel(hidden_states, W1, b1, W2, b2):
    B, S, H = hidden_states.shape
    E = W2.shape[0]
    k = 2
    capacity = int(B * S * 1.5 * k / E)
    N = B * S

    x = hidden_states.reshape(N, H)
    w1t = W1.T
    w2t = W2.T
    b1r = b1.reshape(1, H)
    b2r = b2.reshape(1, E)

    T1 = 256
    nr = N // T1
    TW = 128
    nw = N // TW
    last_r = nr - 1
    L = 128

    probs, aux, dispatch, combine = pl.pallas_call(
        functools.partial(_fused_kernel, nr=nr, t1=T1, tw=TW, cap=capacity),
        grid=(nr + nw,),
        in_specs=[
            pl.BlockSpec((T1, H), lambda i: (jnp.minimum(i, last_r), 0)),
            pl.BlockSpec((H, H), lambda i: (0, 0)),
            pl.BlockSpec((1, H), lambda i: (0, 0)),
            pl.BlockSpec((H, E), lambda i: (0, 0)),
            pl.BlockSpec((1, E), lambda i: (0, 0)),
        ],
        out_specs=[
            pl.BlockSpec((T1, E), lambda i: (jnp.minimum(i, last_r), 0)),
            pl.BlockSpec((1, 1), lambda i: (0, 0)),
            pl.BlockSpec(memory_space=pl.ANY),
            pl.BlockSpec(memory_space=pl.ANY),
        ],
        out_shape=[
            jax.ShapeDtypeStruct((N, E), jnp.float32),
            jax.ShapeDtypeStruct((1, 1), jnp.float32),
            jax.ShapeDtypeStruct((N, E, capacity), jnp.float32),
            jax.ShapeDtypeStruct((N, E, capacity), jnp.float32),
        ],
        scratch_shapes=[
            pltpu.VMEM((N, E), jnp.float32),
            pltpu.VMEM((N, E), jnp.float32),
            pltpu.VMEM((N, E), jnp.float32),
            pltpu.VMEM((N, E), jnp.float32),
            pltpu.VMEM((1, E), jnp.float32),
            pltpu.VMEM((1, E), jnp.float32),
            pltpu.VMEM((TW, E, capacity), jnp.float32),
            pltpu.VMEM((2, TW, E, L), jnp.float32),
            pltpu.VMEM((2, TW, E, L), jnp.float32),
            pltpu.SemaphoreType.DMA((16,)),
            pltpu.SemaphoreType.DMA,
        ],
    )(x, w1t, b1r, w2t, b2r)

    return (dispatch.reshape(B, S, E, capacity),
            combine.reshape(B, S, E, capacity),
            probs.reshape(B, S, E),
            aux[0, 0])


# speedup vs baseline: 1.1028x; 1.1028x over previous
import functools

import jax
import jax.numpy as jnp
from jax.experimental import pallas as pl
from jax.experimental.pallas import tpu as pltpu


def _fused_kernel(x_ref, w1t_ref, b1_ref, w2t_ref, b2_ref,
                  probs_ref, aux_ref, disp_ref, comb_ref,
                  oh0_s, oh1_s, ct1_s, ct2_s, av_s, ps_s,
                  z_s, hd_s, hc_s, tail_sem, head_sem,
                  *, nr, t1, tw, cap):
    i = pl.program_id(0)
    L = 128
    nw = pl.num_programs(0) - nr

    @pl.when(i == 0)
    def _zinit():
        z_s[...] = jnp.zeros_like(z_s)

    @pl.when(i < nr)
    def _route():
        x = x_ref[...]
        h = jnp.dot(x, w1t_ref[...], preferred_element_type=jnp.float32)
        h = jnp.maximum(h + b1_ref[...], 0.0)
        logits = jnp.dot(h, w2t_ref[...], preferred_element_type=jnp.float32)
        logits = logits + b2_ref[...]

        m = jnp.max(logits, axis=-1, keepdims=True)
        ex = jnp.exp(logits - m)
        probs = ex / jnp.sum(ex, axis=-1, keepdims=True)
        probs_ref[...] = probs

        S, E = probs.shape
        iota = jax.lax.broadcasted_iota(jnp.int32, (S, E), 1)
        e0 = jnp.argmax(probs, axis=-1)
        oh0 = iota == e0[:, None]
        p0 = jnp.max(probs, axis=-1, keepdims=True)
        masked = jnp.where(oh0, -1.0, probs)
        e1 = jnp.argmax(masked, axis=-1)
        oh1 = iota == e1[:, None]
        p1 = jnp.max(masked, axis=-1, keepdims=True)
        tot = p0 + p1

        oh0f = oh0.astype(jnp.float32)
        oh1f = oh1.astype(jnp.float32)
        rows = pl.ds(i * t1, t1)
        oh0_s[rows, :] = oh0f
        oh1_s[rows, :] = oh1f
        ct1_s[rows, :] = oh0f * (p0 / tot)
        ct2_s[rows, :] = oh1f * (p1 / tot)

        blk_a = jnp.max(oh0f, axis=0, keepdims=True)
        blk_s = jnp.sum(probs, axis=0, keepdims=True)

        @pl.when(i == 0)
        def _init():
            av_s[...] = blk_a
            ps_s[...] = blk_s

        @pl.when(i > 0)
        def _acc():
            av_s[...] = jnp.maximum(av_s[...], blk_a)
            ps_s[...] = ps_s[...] + blk_s

        @pl.when(i == nr - 1)
        def _fin():
            mean_probs = ps_s[...] / (t1 * nr)
            aux_ref[...] = jnp.sum(
                mean_probs * jnp.log(mean_probs * E + 1e-9),
                axis=-1, keepdims=True)

    @pl.when(jnp.logical_and(i >= 1, i <= nr))
    def _tail_start():
        w0 = (i - 1) * 2
        for j in range(2):
            rows = pl.ds((w0 + j) * tw, tw)
            lanes = pl.ds(L, cap - L)
            pltpu.make_async_copy(
                z_s, disp_ref.at[rows, :, lanes], tail_sem).start()
            pltpu.make_async_copy(
                z_s, comb_ref.at[rows, :, lanes], tail_sem).start()

    @pl.when(i >= nr)
    def _write():
        w = i - nr
        tlanes = pl.ds(L, cap - L)
        trows = pl.ds(w * tw, tw)
        pltpu.make_async_copy(
            z_s, disp_ref.at[trows, :, tlanes], tail_sem).wait()
        pltpu.make_async_copy(
            z_s, comb_ref.at[trows, :, tlanes], tail_sem).wait()
        p = jax.lax.rem(w, 2)
        rows = pl.ds(w * tw, tw)
        a = av_s[...]
        oh0 = oh0_s[rows, :]
        oh1 = oh1_s[rows, :]
        ct1 = ct1_s[rows, :]
        ct2 = ct2_s[rows, :]
        sec1 = oh1 * a
        d0 = (oh0 + (oh1 - sec1))[:, :, None]
        d1 = sec1[:, :, None]
        c1b = ct2 * a
        c0 = (ct1 + (ct2 - c1b))[:, :, None]
        c1 = c1b[:, :, None]
        ci = jax.lax.broadcasted_iota(jnp.int32, (tw, 8, L), 2)
        is0 = ci == 0
        is1 = ci == 1
        hd_s[p] = jnp.where(is0, d0, jnp.where(is1, d1, 0.0))
        hc_s[p] = jnp.where(is0, c0, jnp.where(is1, c1, 0.0))
        lanes = pl.ds(0, L)
        pltpu.make_async_copy(
            hd_s.at[p], disp_ref.at[rows, :, lanes], head_sem).start()
        pltpu.make_async_copy(
            hc_s.at[p], comb_ref.at[rows, :, lanes], head_sem).start()

        @pl.when(w > 0)
        def _head_wait_prev():
            q = jax.lax.rem(w - 1, 2)
            prows = pl.ds((w - 1) * tw, tw)
            pltpu.make_async_copy(
                hd_s.at[q], disp_ref.at[prows, :, lanes], head_sem).wait()
            pltpu.make_async_copy(
                hc_s.at[q], comb_ref.at[prows, :, lanes], head_sem).wait()

        @pl.when(w == nw - 1)
        def _head_wait_last():
            pltpu.make_async_copy(
                hd_s.at[p], disp_ref.at[rows, :, lanes], head_sem).wait()
            pltpu.make_async_copy(
                hc_s.at[p], comb_ref.at[rows, :, lanes], head_sem).wait()


def kernel(hidden_states, W1, b1, W2, b2):
    B, S, H = hidden_states.shape
    E = W2.shape[0]
    k = 2
    capacity = int(B * S * 1.5 * k / E)
    N = B * S

    x = hidden_states.reshape(N, H)
    w1t = W1.T
    w2t = W2.T
    b1r = b1.reshape(1, H)
    b2r = b2.reshape(1, E)

    T1 = 256
    nr = N // T1
    TW = 128
    nw = N // TW
    last_r = nr - 1
    L = 128

    probs, aux, dispatch, combine = pl.pallas_call(
        functools.partial(_fused_kernel, nr=nr, t1=T1, tw=TW, cap=capacity),
        grid=(nr + nw,),
        in_specs=[
            pl.BlockSpec((T1, H), lambda i: (jnp.minimum(i, last_r), 0)),
            pl.BlockSpec((H, H), lambda i: (0, 0)),
            pl.BlockSpec((1, H), lambda i: (0, 0)),
            pl.BlockSpec((H, E), lambda i: (0, 0)),
            pl.BlockSpec((1, E), lambda i: (0, 0)),
        ],
        out_specs=[
            pl.BlockSpec((T1, E), lambda i: (jnp.minimum(i, last_r), 0)),
            pl.BlockSpec((1, 1), lambda i: (0, 0)),
            pl.BlockSpec(memory_space=pl.ANY),
            pl.BlockSpec(memory_space=pl.ANY),
        ],
        out_shape=[
            jax.ShapeDtypeStruct((N, E), jnp.float32),
            jax.ShapeDtypeStruct((1, 1), jnp.float32),
            jax.ShapeDtypeStruct((N, E, capacity), jnp.float32),
            jax.ShapeDtypeStruct((N, E, capacity), jnp.float32),
        ],
        scratch_shapes=[
            pltpu.VMEM((N, E), jnp.float32),
            pltpu.VMEM((N, E), jnp.float32),
            pltpu.VMEM((N, E), jnp.float32),
            pltpu.VMEM((N, E), jnp.float32),
            pltpu.VMEM((1, E), jnp.float32),
            pltpu.VMEM((1, E), jnp.float32),
            pltpu.VMEM((TW, E, capacity - L), jnp.float32),
            pltpu.VMEM((2, TW, E, L), jnp.float32),
            pltpu.VMEM((2, TW, E, L), jnp.float32),
            pltpu.SemaphoreType.DMA,
            pltpu.SemaphoreType.DMA,
        ],
    )(x, w1t, b1r, w2t, b2r)

    return (dispatch.reshape(B, S, E, capacity),
            combine.reshape(B, S, E, capacity),
            probs.reshape(B, S, E),
            aux[0, 0])
